# baseline (speedup 1.0000x reference)
import jax
import jax.numpy as jnp
from jax.experimental import pallas as pl
from jax.experimental.pallas import tpu as pltpu


def _manual_kernel(idx_ref, w_ref, lam_ref, x_hbm, o_hbm,
                   in_buf, out_buf, at_ref, in_sem, out_sem):
    b, h = in_buf.shape[0], in_buf.shape[1]

    in_copies = []
    for i in range(b):
        cp = pltpu.make_async_copy(
            x_hbm.at[pl.ds(i * h, h)], in_buf.at[i], in_sem.at[i])
        cp.start()
        in_copies.append(cp)

    k_fan = idx_ref.shape[0]
    row = jax.lax.broadcasted_iota(jnp.int32, (h, h), 0)
    mt = jnp.zeros((h, h), dtype=jnp.float32)
    for k in range(k_fan):
        hit = (row == idx_ref[k:k + 1, :]).astype(jnp.float32)
        mt = mt + w_ref[0, k] * hit
    mt5 = mt
    for _ in range(4):
        mt5 = jnp.dot(mt, mt5, preferred_element_type=jnp.float32)
    lam = lam_ref[0, 0]
    at_ref[...] = (lam * lam * lam * lam * lam) * mt5

    wfull, c = in_buf.shape[2], in_buf.shape[3]
    whalf = wfull // 8
    out_copies = []
    for i in range(b):
        in_copies[i].wait()
        for q in range(8):
            xq = in_buf[i, :, q * whalf:(q + 1) * whalf, :]
            x2 = xq.reshape(h, whalf * c)
            ob = jax.lax.dot_general(
                at_ref[...], x2, (((0,), (0,)), ((), ())),
                preferred_element_type=jnp.float32)
            out_buf[i, :, q * whalf:(q + 1) * whalf, :] = ob.reshape(h, whalf, c)
            cp = pltpu.make_async_copy(
                out_buf.at[i, :, pl.ds(q * whalf, whalf)],
                o_hbm.at[pl.ds(i * h, h), pl.ds(q * whalf, whalf)],
                out_sem.at[i, q])
            cp.start()
            out_copies.append(cp)
    for cp in out_copies:
        cp.wait()


def kernel(inputs, ind1, w1, lambda1):
    b, h, w, c = inputs.shape
    k_fan = ind1.shape[0]

    idx = ind1[..., 0].astype(jnp.int32)
    wv = w1.reshape(1, k_fan).astype(jnp.float32)
    lam = lambda1.reshape(1, 1).astype(jnp.float32)

    x3 = inputs.reshape(b * h, w, c)
    out3 = pl.pallas_call(
        _manual_kernel,
        in_specs=[
            pl.BlockSpec(memory_space=pltpu.VMEM),
            pl.BlockSpec(memory_space=pltpu.SMEM),
            pl.BlockSpec(memory_space=pltpu.SMEM),
            pl.BlockSpec(memory_space=pl.ANY),
        ],
        out_specs=pl.BlockSpec(memory_space=pl.ANY),
        out_shape=jax.ShapeDtypeStruct((b * h, w, c), jnp.float32),
        scratch_shapes=[
            pltpu.VMEM((b, h, w, c), jnp.float32),
            pltpu.VMEM((b, h, w, c), jnp.float32),
            pltpu.VMEM((h, h), jnp.float32),
            pltpu.SemaphoreType.DMA((b,)),
            pltpu.SemaphoreType.DMA((b, 8)),
        ],
    )(idx, wv, lam, x3)

    return out3.reshape(b, h, w, c)
